# baseline (speedup 1.0000x reference)
import functools

import jax
import jax.numpy as jnp
from jax import lax
from jax.experimental import pallas as pl
from jax.experimental.pallas import tpu as pltpu
from jax.experimental.pallas import tpu_sc as plsc

_INTERPRET = False

KNN = 3
NEG_INF = float("-inf")


def _insert(rs, ks, r4, bv, bk):
    r1, r2, r3 = rs
    k1, k2, k3 = ks
    g1 = bv > r1
    vd = jnp.where(g1, r1, bv)
    kd = jnp.where(g1, k1, bk)
    r1 = jnp.where(g1, bv, r1)
    k1 = jnp.where(g1, bk, k1)
    g2 = vd > r2
    vd2 = jnp.where(g2, r2, vd)
    kd2 = jnp.where(g2, k2, kd)
    r2 = jnp.where(g2, vd, r2)
    k2 = jnp.where(g2, kd, k2)
    g3 = vd2 > r3
    vd3 = jnp.where(g3, r3, vd2)
    r3 = jnp.where(g3, vd2, r3)
    k3 = jnp.where(g3, kd2, k3)
    r4 = jnp.maximum(r4, vd3)
    return (r1, r2, r3), (k1, k2, k3), r4


def _knn_idx_body(x_ref, t4_ref, w1n_ref, w1c_ref,
                  idx_ref, p_ref, q_ref, *, rb, n):
    b = pl.program_id(0)
    jb = pl.program_id(1)

    x0j = x_ref[0, 0:1, :]
    x1j = x_ref[0, 1:2, :]
    xi = t4_ref[0, pl.ds(jb * rb, rb), :][:, 0:2]
    xi0 = xi[:, 0:1]
    xi1 = xi[:, 1:2]

    p_ref[:, 0:32] = jnp.dot(xi, w1n_ref[...], preferred_element_type=jnp.float32)
    p_ref[:, 32:128] = jnp.zeros((rb, 96), jnp.float32)
    q_ref[...] = jnp.dot(xi, w1c_ref[...], preferred_element_type=jnp.float32)

    xxj = x0j * x0j + x1j * x1j
    xxi = xi0 * xi0 + xi1 * xi1
    inner = jnp.dot(-2.0 * xi, x_ref[0], preferred_element_type=jnp.float32)
    negxxj = 0.0 - xxj

    cw = 128
    neg = jnp.full((rb, cw), NEG_INF, jnp.float32)
    rs = (neg, neg, neg)
    ks = (neg, neg, neg)
    r4 = neg
    lanekey = jax.lax.broadcasted_iota(jnp.int32, (1, cw), 1).astype(jnp.float32)
    for c in range(n // cw):
        sl = slice(c * cw, (c + 1) * cw)
        pdc = (negxxj[:, sl] - inner[:, sl]) - xxi
        keyc = lanekey + float(c * cw)
        rs, ks, r4 = _insert(rs, ks, r4, pdc, keyc)
    w = cw
    while w > 1:
        hw = w // 2
        ars = tuple(r[:, :hw] for r in rs)
        aks = tuple(k[:, :hw] for k in ks)
        ar4 = r4[:, :hw]
        brs = tuple(r[:, hw:w] for r in rs)
        bks = tuple(k[:, hw:w] for k in ks)
        br4 = r4[:, hw:w]
        rs, ks, r4 = ars, aks, jnp.maximum(ar4, br4)
        for t in range(3):
            rs, ks, r4 = _insert(rs, ks, r4, brs[t], bks[t])
        w = hw
    base = b * n

    tie = jnp.max(jnp.where(rs[2] == r4, 1.0, 0.0)) > 0.5
    for kk in range(KNN):
        idx_ref[:, kk:kk + 1] = ks[kk].astype(jnp.int32) + base

    @pl.when(tie)
    def _():
        iota = jax.lax.broadcasted_iota(jnp.int32, (rb, n), 1).astype(jnp.float32)
        pdl = (negxxj - inner) - xxi
        for kk in range(KNN):
            m = jnp.max(pdl, axis=1, keepdims=True)
            isel = jnp.min(jnp.where(pdl == m, iota, float(n)),
                           axis=1, keepdims=True)
            idx_ref[:, kk:kk + 1] = isel.astype(jnp.int32) + base
            if kk + 1 < KNN:
                pdl = jnp.where(iota == isel, NEG_INF, pdl)


def _knn_idx(x, t4, w1n, w1c, rb):
    b_, d_, n = x.shape
    nb = n // rb
    body = functools.partial(_knn_idx_body, rb=rb, n=n)
    return pl.pallas_call(
        body,
        grid=(b_, nb),
        in_specs=[
            pl.BlockSpec((1, 2, n), lambda b, j: (b, 0, 0)),
            pl.BlockSpec((1, n, 4), lambda b, j: (b, 0, 0)),
            pl.BlockSpec((2, 32), lambda b, j: (0, 0)),
            pl.BlockSpec((2, 32), lambda b, j: (0, 0)),
        ],
        out_specs=[
            pl.BlockSpec((rb, KNN), lambda b, j, nb=nb: (b * nb + j, 0)),
            pl.BlockSpec((rb, 128), lambda b, j, nb=nb: (b * nb + j, 0)),
            pl.BlockSpec((rb, 32), lambda b, j, nb=nb: (b * nb + j, 0)),
        ],
        out_shape=[
            jax.ShapeDtypeStruct((b_ * n, KNN), jnp.int32),
            jax.ShapeDtypeStruct((b_ * n, 128), jnp.float32),
            jax.ShapeDtypeStruct((b_ * n, 32), jnp.float32),
        ],
        interpret=_INTERPRET,
    )(x, t4, w1n, w1c)


def _sc_gather(table, idx_flat):
    e_, dd = idx_flat.shape[0], table.shape[1]
    info = plsc.get_sparse_core_info()
    nc, ns = info.num_cores, info.num_subcores
    nw = nc * ns
    b_per_w = e_ // nw
    nch = 2
    b_per_c = b_per_w // nch
    mesh = plsc.VectorSubcoreMesh(core_axis_name="c", subcore_axis_name="s")

    @functools.partial(
        pl.kernel, mesh=mesh,
        out_type=jax.ShapeDtypeStruct((e_, dd), jnp.float32),
        scratch_types=[
            pltpu.VMEM((b_per_c,), jnp.int32),
            pltpu.VMEM((b_per_c, dd), jnp.float32),
            pltpu.SemaphoreType.DMA,
        ],
    )
    def k(table_hbm, idx_hbm, out_hbm, idx_v, rows_v, sem):
        wid = lax.axis_index("s") * nc + lax.axis_index("c")
        for c in range(nch):
            base = wid * b_per_w + c * b_per_c
            pltpu.sync_copy(idx_hbm.at[pl.ds(base, b_per_c)], idx_v)
            pltpu.async_copy(table_hbm.at[idx_v], rows_v, sem).wait()
            pltpu.sync_copy(rows_v, out_hbm.at[pl.ds(base, b_per_c)])

    return k(table, idx_flat)


def _conv1_body(g_ref, q_ref, h1_ref, s_ref, ss_ref):
    j = pl.program_id(0)
    q = q_ref[...]
    s_loc = jnp.zeros((1, 32), jnp.float32)
    ss_loc = jnp.zeros((1, 32), jnp.float32)
    for kk in range(KNN):
        h1k = g_ref[kk][:, 0:32] + q
        h1_ref[kk] = h1k
        s_loc = s_loc + jnp.sum(h1k, axis=0, keepdims=True)
        ss_loc = ss_loc + jnp.sum(h1k * h1k, axis=0, keepdims=True)

    @pl.when(j == 0)
    def _():
        s_ref[...] = jnp.zeros_like(s_ref)
        ss_ref[...] = jnp.zeros_like(ss_ref)

    s_ref[...] += s_loc
    ss_ref[...] += ss_loc


def _conv1(gath, q, pr):
    p = q.shape[0]
    nb = p // pr
    return pl.pallas_call(
        _conv1_body,
        grid=(nb,),
        in_specs=[
            pl.BlockSpec((KNN, pr, 128), lambda j: (0, j, 0)),
            pl.BlockSpec((pr, 32), lambda j: (j, 0)),
        ],
        out_specs=[
            pl.BlockSpec((KNN, pr, 32), lambda j: (0, j, 0)),
            pl.BlockSpec((1, 32), lambda j: (0, 0)),
            pl.BlockSpec((1, 32), lambda j: (0, 0)),
        ],
        out_shape=[
            jax.ShapeDtypeStruct((KNN, p, 32), jnp.float32),
            jax.ShapeDtypeStruct((1, 32), jnp.float32),
            jax.ShapeDtypeStruct((1, 32), jnp.float32),
        ],
        interpret=_INTERPRET,
    )(gath, q)


def _stage_body(h_ref, sc_ref, sh_ref, wt_ref, xp_ref, hn_ref, s_ref, ss_ref,
                *, cout):
    j = pl.program_id(0)
    sc = sc_ref[...]
    sh = sh_ref[...]
    a = [jnp.maximum(h_ref[kk] * sc + sh, 0.0) for kk in range(KNN)]
    xp_ref[...] = jnp.maximum(jnp.maximum(a[0], a[1]), a[2])

    s_loc = jnp.zeros((1, cout), jnp.float32)
    ss_loc = jnp.zeros((1, cout), jnp.float32)
    for kk in range(KNN):
        hn = jnp.dot(a[kk], wt_ref[...], preferred_element_type=jnp.float32)
        hn_ref[kk] = hn
        s_loc = s_loc + jnp.sum(hn, axis=0, keepdims=True)
        ss_loc = ss_loc + jnp.sum(hn * hn, axis=0, keepdims=True)

    @pl.when(j == 0)
    def _():
        s_ref[...] = jnp.zeros_like(s_ref)
        ss_ref[...] = jnp.zeros_like(ss_ref)

    s_ref[...] += s_loc
    ss_ref[...] += ss_loc


def _stage(h, scale, shift, wt, pr):
    p = h.shape[1]
    cin = h.shape[2]
    cout = wt.shape[1]
    nb = p // pr
    body = functools.partial(_stage_body, cout=cout)
    return pl.pallas_call(
        body,
        grid=(nb,),
        in_specs=[
            pl.BlockSpec((KNN, pr, cin), lambda j: (0, j, 0)),
            pl.BlockSpec((1, cin), lambda j: (0, 0)),
            pl.BlockSpec((1, cin), lambda j: (0, 0)),
            pl.BlockSpec((cin, cout), lambda j: (0, 0)),
        ],
        out_specs=[
            pl.BlockSpec((pr, cin), lambda j: (j, 0)),
            pl.BlockSpec((KNN, pr, cout), lambda j: (0, j, 0)),
            pl.BlockSpec((1, cout), lambda j: (0, 0)),
            pl.BlockSpec((1, cout), lambda j: (0, 0)),
        ],
        out_shape=[
            jax.ShapeDtypeStruct((p, cin), jnp.float32),
            jax.ShapeDtypeStruct((KNN, p, cout), jnp.float32),
            jax.ShapeDtypeStruct((1, cout), jnp.float32),
            jax.ShapeDtypeStruct((1, cout), jnp.float32),
        ],
        interpret=_INTERPRET,
    )(h, scale, shift, wt)


def _stage4_body(h_ref, sc_ref, sh_ref, wt_ref, xp_ref, s_ref, ss_ref):
    j = pl.program_id(0)
    sc = sc_ref[...]
    sh = sh_ref[...]
    a = [jnp.maximum(h_ref[kk] * sc + sh, 0.0) for kk in range(KNN)]
    xp_ref[...] = jnp.maximum(jnp.maximum(a[0], a[1]), a[2])

    s_loc = jnp.zeros((1, 256), jnp.float32)
    ss_loc = jnp.zeros((1, 256), jnp.float32)
    for kk in range(KNN):
        hn = jnp.dot(a[kk], wt_ref[...], preferred_element_type=jnp.float32)
        s_loc = s_loc + jnp.sum(hn, axis=0, keepdims=True)
        ss_loc = ss_loc + jnp.sum(hn * hn, axis=0, keepdims=True)

    @pl.when(j == 0)
    def _():
        s_ref[...] = jnp.zeros_like(s_ref)
        ss_ref[...] = jnp.zeros_like(ss_ref)

    s_ref[...] += s_loc
    ss_ref[...] += ss_loc


def _stage4(h3, scale, shift, w4t, pr):
    p = h3.shape[1]
    nb = p // pr
    return pl.pallas_call(
        _stage4_body,
        grid=(nb,),
        in_specs=[
            pl.BlockSpec((KNN, pr, 128), lambda j: (0, j, 0)),
            pl.BlockSpec((1, 128), lambda j: (0, 0)),
            pl.BlockSpec((1, 128), lambda j: (0, 0)),
            pl.BlockSpec((128, 256), lambda j: (0, 0)),
        ],
        out_specs=[
            pl.BlockSpec((pr, 128), lambda j: (j, 0)),
            pl.BlockSpec((1, 256), lambda j: (0, 0)),
            pl.BlockSpec((1, 256), lambda j: (0, 0)),
        ],
        out_shape=[
            jax.ShapeDtypeStruct((p, 128), jnp.float32),
            jax.ShapeDtypeStruct((1, 256), jnp.float32),
            jax.ShapeDtypeStruct((1, 256), jnp.float32),
        ],
        interpret=_INTERPRET,
    )(h3, scale, shift, w4t)


def _final_conv_body(h_ref, sc3_ref, sh3_ref, w4t_ref, sc4_ref, sh4_ref,
                     x1_ref, x2_ref, x3_ref,
                     w5a_ref, w5b_ref, w5c_ref, w5d_ref,
                     h5_ref, s_ref, ss_ref):
    j = pl.program_id(0)
    sc3 = sc3_ref[...]
    sh3 = sh3_ref[...]
    sc4 = sc4_ref[...]
    sh4 = sh4_ref[...]
    x4 = None
    for kk in range(KNN):
        a3 = jnp.maximum(h_ref[kk] * sc3 + sh3, 0.0)
        h4 = jnp.dot(a3, w4t_ref[...], preferred_element_type=jnp.float32)
        a4 = jnp.maximum(h4 * sc4 + sh4, 0.0)
        x4 = a4 if x4 is None else jnp.maximum(x4, a4)

    h5 = (jnp.dot(x1_ref[...], w5a_ref[...], preferred_element_type=jnp.float32)
          + jnp.dot(x2_ref[...], w5b_ref[...], preferred_element_type=jnp.float32)
          + jnp.dot(x3_ref[...], w5c_ref[...], preferred_element_type=jnp.float32)
          + jnp.dot(x4, w5d_ref[...], preferred_element_type=jnp.float32))
    h5_ref[...] = h5

    @pl.when(j == 0)
    def _():
        s_ref[...] = jnp.zeros_like(s_ref)
        ss_ref[...] = jnp.zeros_like(ss_ref)

    s_ref[...] += jnp.sum(h5, axis=0, keepdims=True)
    ss_ref[...] += jnp.sum(h5 * h5, axis=0, keepdims=True)


def _final_conv(h3, sc3, sh3, w4t, sc4, sh4, x1, x2, x3,
                w5a, w5b, w5c, w5d, pr):
    p = h3.shape[1]
    nb = p // pr
    return pl.pallas_call(
        _final_conv_body,
        grid=(nb,),
        in_specs=[
            pl.BlockSpec((KNN, pr, 128), lambda j: (0, j, 0)),
            pl.BlockSpec((1, 128), lambda j: (0, 0)),
            pl.BlockSpec((1, 128), lambda j: (0, 0)),
            pl.BlockSpec((128, 256), lambda j: (0, 0)),
            pl.BlockSpec((1, 256), lambda j: (0, 0)),
            pl.BlockSpec((1, 256), lambda j: (0, 0)),
            pl.BlockSpec((pr, 32), lambda j: (j, 0)),
            pl.BlockSpec((pr, 64), lambda j: (j, 0)),
            pl.BlockSpec((pr, 128), lambda j: (j, 0)),
            pl.BlockSpec((32, 512), lambda j: (0, 0)),
            pl.BlockSpec((64, 512), lambda j: (0, 0)),
            pl.BlockSpec((128, 512), lambda j: (0, 0)),
            pl.BlockSpec((256, 512), lambda j: (0, 0)),
        ],
        out_specs=[
            pl.BlockSpec((pr, 512), lambda j: (j, 0)),
            pl.BlockSpec((1, 512), lambda j: (0, 0)),
            pl.BlockSpec((1, 512), lambda j: (0, 0)),
        ],
        out_shape=[
            jax.ShapeDtypeStruct((p, 512), jnp.float32),
            jax.ShapeDtypeStruct((1, 512), jnp.float32),
            jax.ShapeDtypeStruct((1, 512), jnp.float32),
        ],
        interpret=_INTERPRET,
    )(h3, sc3, sh3, w4t, sc4, sh4, x1, x2, x3, w5a, w5b, w5c, w5d)


def _out_body(h5_ref, sc_ref, sh_ref, o_ref):
    a = jnp.maximum(h5_ref[...] * sc_ref[...] + sh_ref[...], 0.0)
    o_ref[0] = a.T


def _out_pass(h5, scale, shift, b_, n, pr):
    nb = n // pr
    return pl.pallas_call(
        _out_body,
        grid=(b_, nb),
        in_specs=[
            pl.BlockSpec((pr, 512), lambda b, j, nb=nb: (b * nb + j, 0)),
            pl.BlockSpec((1, 512), lambda b, j: (0, 0)),
            pl.BlockSpec((1, 512), lambda b, j: (0, 0)),
        ],
        out_specs=pl.BlockSpec((1, 512, pr), lambda b, j: (b, 0, j)),
        out_shape=jax.ShapeDtypeStruct((b_, 512, n), jnp.float32),
        interpret=_INTERPRET,
    )(h5, scale, shift)


def _bn_coeffs(s, ss, m, g, b, eps=1e-5):
    mean = s / m
    var = jnp.maximum(ss / m - mean * mean, 0.0)
    scale = g[None, :] / jnp.sqrt(var + eps)
    shift = b[None, :] - mean * scale
    return scale, shift


def kernel(x, W1, W2, W3, W4, W5, g1, b1, g2, b2, g3, b3, g4, b4, g5, b5):
    b_, d_, n = x.shape
    xt = jnp.swapaxes(x, 1, 2)
    t4 = jnp.concatenate(
        [xt, jnp.ones((b_, n, 1), jnp.float32),
         jnp.zeros((b_, n, 1), jnp.float32)], axis=2)
    w1n = W1[:, :2].T
    w1c = W1[:, 2:].T

    rb = 512
    pr = 1024
    p = b_ * n
    m_edge = float(p * KNN)
    m_pt = float(p)

    idx, ptab, qtab = _knn_idx(x, t4, w1n, w1c, rb)
    idx_flat = idx.T.reshape(-1)
    gath = _sc_gather(ptab, idx_flat).reshape(KNN, p, 128)
    h1, s1, ss1 = _conv1(gath, qtab, pr)
    sc1, sh1 = _bn_coeffs(s1, ss1, m_edge, g1, b1)

    x1, h2, s2, ss2 = _stage(h1, sc1, sh1, W2.T, pr)
    sc2, sh2 = _bn_coeffs(s2, ss2, m_edge, g2, b2)

    x2, h3, s3, ss3 = _stage(h2, sc2, sh2, W3.T, pr)
    sc3, sh3 = _bn_coeffs(s3, ss3, m_edge, g3, b3)

    w4t = W4.T
    x3, s4, ss4 = _stage4(h3, sc3, sh3, w4t, pr)
    sc4, sh4 = _bn_coeffs(s4, ss4, m_edge, g4, b4)

    w5t = W5.T
    h5, s5, ss5 = _final_conv(h3, sc3, sh3, w4t, sc4, sh4, x1, x2, x3,
                              w5t[:32], w5t[32:96], w5t[96:224], w5t[224:],
                              pr)
    sc5, sh5 = _bn_coeffs(s5, ss5, m_pt, g5, b5)

    return _out_pass(h5, sc5, sh5, b_, n, pr)


# speedup vs baseline: 1.2623x; 1.2623x over previous
import functools

import jax
import jax.numpy as jnp
from jax import lax
from jax.experimental import pallas as pl
from jax.experimental.pallas import tpu as pltpu
from jax.experimental.pallas import tpu_sc as plsc

_INTERPRET = False

KNN = 3
NEG_INF = float("-inf")


def _knn_idx_body(x_ref, t4_ref, w1n_ref, w1c_ref,
                  idx_ref, p_ref, q_ref, *, rb, n):
    b = pl.program_id(0)
    jb = pl.program_id(1)

    x0j = x_ref[0, 0:1, :]
    x1j = x_ref[0, 1:2, :]
    xi = t4_ref[0, pl.ds(jb * rb, rb), :][:, 0:2]
    xi0 = xi[:, 0:1]
    xi1 = xi[:, 1:2]

    p_ref[:, 0:32] = jnp.dot(xi, w1n_ref[...], preferred_element_type=jnp.float32)
    p_ref[:, 32:128] = jnp.zeros((rb, 96), jnp.float32)
    q_ref[...] = jnp.dot(xi, w1c_ref[...], preferred_element_type=jnp.float32)

    xxj = x0j * x0j + x1j * x1j
    xxi = xi0 * xi0 + xi1 * xi1
    inner = jnp.dot(-2.0 * xi, x_ref[0], preferred_element_type=jnp.float32)
    negxxj = 0.0 - xxj

    cw = 128
    neg = jnp.full((rb, cw), NEG_INF, jnp.float32)
    r1 = r2 = r3 = r4 = neg
    for c in range(n // cw):
        sl = slice(c * cw, (c + 1) * cw)
        pdc = (negxxj[:, sl] - inner[:, sl]) - xxi
        t = jnp.minimum(r1, pdc)
        r1 = jnp.maximum(r1, pdc)
        t2 = jnp.minimum(r2, t)
        r2 = jnp.maximum(r2, t)
        t3 = jnp.minimum(r3, t2)
        r3 = jnp.maximum(r3, t2)
        r4 = jnp.maximum(r4, t3)
    w = cw
    while w > 1:
        hw = w // 2
        a1, b1 = r1[:, :hw], r1[:, hw:w]
        a2, b2 = r2[:, :hw], r2[:, hw:w]
        a3, b3 = r3[:, :hw], r3[:, hw:w]
        a4, b4 = r4[:, :hw], r4[:, hw:w]
        r1 = jnp.maximum(a1, b1)
        t = jnp.minimum(a1, b1)
        u = jnp.maximum(a2, b2)
        r2 = jnp.maximum(t, u)
        r3 = jnp.maximum(jnp.minimum(t, u),
                         jnp.maximum(jnp.minimum(a2, b2),
                                     jnp.maximum(a3, b3)))
        r4 = jnp.maximum(
            jnp.maximum(b4, a4),
            jnp.maximum(jnp.minimum(a1, b3),
                        jnp.maximum(jnp.minimum(a2, b2),
                                    jnp.minimum(a3, b1))))
        w = hw
    base = b * n
    ms = (r1, r2, r3)

    tie = jnp.max(jnp.where((r1 == r2) | (r2 == r3) | (r3 == r4),
                            1.0, 0.0)) > 0.5

    iota = jax.lax.broadcasted_iota(jnp.int32, (rb, n), 1).astype(jnp.float32)
    pdfull = (negxxj - inner) - xxi
    for kk in range(KNN):
        isel = jnp.min(jnp.where(pdfull == ms[kk], iota, float(n)),
                       axis=1, keepdims=True)
        idx_ref[:, kk:kk + 1] = isel.astype(jnp.int32) + base

    @pl.when(tie)
    def _():
        pdl = pdfull
        for kk in range(KNN):
            m = jnp.max(pdl, axis=1, keepdims=True)
            isel = jnp.min(jnp.where(pdl == m, iota, float(n)),
                           axis=1, keepdims=True)
            idx_ref[:, kk:kk + 1] = isel.astype(jnp.int32) + base
            if kk + 1 < KNN:
                pdl = jnp.where(iota == isel, NEG_INF, pdl)


def _knn_idx(x, t4, w1n, w1c, rb):
    b_, d_, n = x.shape
    nb = n // rb
    body = functools.partial(_knn_idx_body, rb=rb, n=n)
    return pl.pallas_call(
        body,
        grid=(b_, nb),
        in_specs=[
            pl.BlockSpec((1, 2, n), lambda b, j: (b, 0, 0)),
            pl.BlockSpec((1, n, 4), lambda b, j: (b, 0, 0)),
            pl.BlockSpec((2, 32), lambda b, j: (0, 0)),
            pl.BlockSpec((2, 32), lambda b, j: (0, 0)),
        ],
        out_specs=[
            pl.BlockSpec((rb, KNN), lambda b, j, nb=nb: (b * nb + j, 0)),
            pl.BlockSpec((rb, 128), lambda b, j, nb=nb: (b * nb + j, 0)),
            pl.BlockSpec((rb, 32), lambda b, j, nb=nb: (b * nb + j, 0)),
        ],
        out_shape=[
            jax.ShapeDtypeStruct((b_ * n, KNN), jnp.int32),
            jax.ShapeDtypeStruct((b_ * n, 128), jnp.float32),
            jax.ShapeDtypeStruct((b_ * n, 32), jnp.float32),
        ],
        interpret=_INTERPRET,
    )(x, t4, w1n, w1c)


def _sc_gather(table, idx_flat):
    e_, dd = idx_flat.shape[0], table.shape[1]
    info = plsc.get_sparse_core_info()
    nc, ns = info.num_cores, info.num_subcores
    nw = nc * ns
    b_per_w = e_ // nw
    nch = 2
    b_per_c = b_per_w // nch
    mesh = plsc.VectorSubcoreMesh(core_axis_name="c", subcore_axis_name="s")

    @functools.partial(
        pl.kernel, mesh=mesh,
        out_type=jax.ShapeDtypeStruct((e_, dd), jnp.float32),
        scratch_types=[
            pltpu.VMEM((b_per_c,), jnp.int32),
            pltpu.VMEM((b_per_c, dd), jnp.float32),
            pltpu.SemaphoreType.DMA,
        ],
    )
    def k(table_hbm, idx_hbm, out_hbm, idx_v, rows_v, sem):
        wid = lax.axis_index("s") * nc + lax.axis_index("c")
        for c in range(nch):
            base = wid * b_per_w + c * b_per_c
            pltpu.sync_copy(idx_hbm.at[pl.ds(base, b_per_c)], idx_v)
            pltpu.async_copy(table_hbm.at[idx_v], rows_v, sem).wait()
            pltpu.sync_copy(rows_v, out_hbm.at[pl.ds(base, b_per_c)])

    return k(table, idx_flat)


def _conv1_body(g_ref, q_ref, h1_ref, s_ref, ss_ref):
    j = pl.program_id(0)
    q = q_ref[...]
    s_loc = jnp.zeros((1, 32), jnp.float32)
    ss_loc = jnp.zeros((1, 32), jnp.float32)
    for kk in range(KNN):
        h1k = g_ref[kk][:, 0:32] + q
        h1_ref[kk] = h1k
        s_loc = s_loc + jnp.sum(h1k, axis=0, keepdims=True)
        ss_loc = ss_loc + jnp.sum(h1k * h1k, axis=0, keepdims=True)

    @pl.when(j == 0)
    def _():
        s_ref[...] = jnp.zeros_like(s_ref)
        ss_ref[...] = jnp.zeros_like(ss_ref)

    s_ref[...] += s_loc
    ss_ref[...] += ss_loc


def _conv1(gath, q, pr):
    p = q.shape[0]
    nb = p // pr
    return pl.pallas_call(
        _conv1_body,
        grid=(nb,),
        in_specs=[
            pl.BlockSpec((KNN, pr, 128), lambda j: (0, j, 0)),
            pl.BlockSpec((pr, 32), lambda j: (j, 0)),
        ],
        out_specs=[
            pl.BlockSpec((KNN, pr, 32), lambda j: (0, j, 0)),
            pl.BlockSpec((1, 32), lambda j: (0, 0)),
            pl.BlockSpec((1, 32), lambda j: (0, 0)),
        ],
        out_shape=[
            jax.ShapeDtypeStruct((KNN, p, 32), jnp.float32),
            jax.ShapeDtypeStruct((1, 32), jnp.float32),
            jax.ShapeDtypeStruct((1, 32), jnp.float32),
        ],
        interpret=_INTERPRET,
    )(gath, q)


def _stage_body(h_ref, sc_ref, sh_ref, wt_ref, xp_ref, hn_ref, s_ref, ss_ref,
                *, cout):
    j = pl.program_id(0)
    sc = sc_ref[...]
    sh = sh_ref[...]
    a = [jnp.maximum(h_ref[kk] * sc + sh, 0.0) for kk in range(KNN)]
    xp_ref[...] = jnp.maximum(jnp.maximum(a[0], a[1]), a[2])

    s_loc = jnp.zeros((1, cout), jnp.float32)
    ss_loc = jnp.zeros((1, cout), jnp.float32)
    for kk in range(KNN):
        hn = jnp.dot(a[kk], wt_ref[...], preferred_element_type=jnp.float32)
        hn_ref[kk] = hn
        s_loc = s_loc + jnp.sum(hn, axis=0, keepdims=True)
        ss_loc = ss_loc + jnp.sum(hn * hn, axis=0, keepdims=True)

    @pl.when(j == 0)
    def _():
        s_ref[...] = jnp.zeros_like(s_ref)
        ss_ref[...] = jnp.zeros_like(ss_ref)

    s_ref[...] += s_loc
    ss_ref[...] += ss_loc


def _stage(h, scale, shift, wt, pr):
    p = h.shape[1]
    cin = h.shape[2]
    cout = wt.shape[1]
    nb = p // pr
    body = functools.partial(_stage_body, cout=cout)
    return pl.pallas_call(
        body,
        grid=(nb,),
        in_specs=[
            pl.BlockSpec((KNN, pr, cin), lambda j: (0, j, 0)),
            pl.BlockSpec((1, cin), lambda j: (0, 0)),
            pl.BlockSpec((1, cin), lambda j: (0, 0)),
            pl.BlockSpec((cin, cout), lambda j: (0, 0)),
        ],
        out_specs=[
            pl.BlockSpec((pr, cin), lambda j: (j, 0)),
            pl.BlockSpec((KNN, pr, cout), lambda j: (0, j, 0)),
            pl.BlockSpec((1, cout), lambda j: (0, 0)),
            pl.BlockSpec((1, cout), lambda j: (0, 0)),
        ],
        out_shape=[
            jax.ShapeDtypeStruct((p, cin), jnp.float32),
            jax.ShapeDtypeStruct((KNN, p, cout), jnp.float32),
            jax.ShapeDtypeStruct((1, cout), jnp.float32),
            jax.ShapeDtypeStruct((1, cout), jnp.float32),
        ],
        interpret=_INTERPRET,
    )(h, scale, shift, wt)


def _stage4_body(h_ref, sc_ref, sh_ref, wt_ref, xp_ref, s_ref, ss_ref):
    j = pl.program_id(0)
    sc = sc_ref[...]
    sh = sh_ref[...]
    a = [jnp.maximum(h_ref[kk] * sc + sh, 0.0) for kk in range(KNN)]
    xp_ref[...] = jnp.maximum(jnp.maximum(a[0], a[1]), a[2])

    s_loc = jnp.zeros((1, 256), jnp.float32)
    ss_loc = jnp.zeros((1, 256), jnp.float32)
    for kk in range(KNN):
        hn = jnp.dot(a[kk], wt_ref[...], preferred_element_type=jnp.float32)
        s_loc = s_loc + jnp.sum(hn, axis=0, keepdims=True)
        ss_loc = ss_loc + jnp.sum(hn * hn, axis=0, keepdims=True)

    @pl.when(j == 0)
    def _():
        s_ref[...] = jnp.zeros_like(s_ref)
        ss_ref[...] = jnp.zeros_like(ss_ref)

    s_ref[...] += s_loc
    ss_ref[...] += ss_loc


def _stage4(h3, scale, shift, w4t, pr):
    p = h3.shape[1]
    nb = p // pr
    return pl.pallas_call(
        _stage4_body,
        grid=(nb,),
        in_specs=[
            pl.BlockSpec((KNN, pr, 128), lambda j: (0, j, 0)),
            pl.BlockSpec((1, 128), lambda j: (0, 0)),
            pl.BlockSpec((1, 128), lambda j: (0, 0)),
            pl.BlockSpec((128, 256), lambda j: (0, 0)),
        ],
        out_specs=[
            pl.BlockSpec((pr, 128), lambda j: (j, 0)),
            pl.BlockSpec((1, 256), lambda j: (0, 0)),
            pl.BlockSpec((1, 256), lambda j: (0, 0)),
        ],
        out_shape=[
            jax.ShapeDtypeStruct((p, 128), jnp.float32),
            jax.ShapeDtypeStruct((1, 256), jnp.float32),
            jax.ShapeDtypeStruct((1, 256), jnp.float32),
        ],
        interpret=_INTERPRET,
    )(h3, scale, shift, w4t)


def _final_conv_body(h_ref, sc3_ref, sh3_ref, w4t_ref, sc4_ref, sh4_ref,
                     x1_ref, x2_ref, x3_ref,
                     w5a_ref, w5b_ref, w5c_ref, w5d_ref,
                     h5_ref, s_ref, ss_ref):
    j = pl.program_id(0)
    sc3 = sc3_ref[...]
    sh3 = sh3_ref[...]
    sc4 = sc4_ref[...]
    sh4 = sh4_ref[...]
    x4 = None
    for kk in range(KNN):
        a3 = jnp.maximum(h_ref[kk] * sc3 + sh3, 0.0)
        h4 = jnp.dot(a3, w4t_ref[...], preferred_element_type=jnp.float32)
        a4 = jnp.maximum(h4 * sc4 + sh4, 0.0)
        x4 = a4 if x4 is None else jnp.maximum(x4, a4)

    h5 = (jnp.dot(x1_ref[...], w5a_ref[...], preferred_element_type=jnp.float32)
          + jnp.dot(x2_ref[...], w5b_ref[...], preferred_element_type=jnp.float32)
          + jnp.dot(x3_ref[...], w5c_ref[...], preferred_element_type=jnp.float32)
          + jnp.dot(x4, w5d_ref[...], preferred_element_type=jnp.float32))
    h5_ref[...] = h5

    @pl.when(j == 0)
    def _():
        s_ref[...] = jnp.zeros_like(s_ref)
        ss_ref[...] = jnp.zeros_like(ss_ref)

    s_ref[...] += jnp.sum(h5, axis=0, keepdims=True)
    ss_ref[...] += jnp.sum(h5 * h5, axis=0, keepdims=True)


def _final_conv(h3, sc3, sh3, w4t, sc4, sh4, x1, x2, x3,
                w5a, w5b, w5c, w5d, pr):
    p = h3.shape[1]
    nb = p // pr
    return pl.pallas_call(
        _final_conv_body,
        grid=(nb,),
        in_specs=[
            pl.BlockSpec((KNN, pr, 128), lambda j: (0, j, 0)),
            pl.BlockSpec((1, 128), lambda j: (0, 0)),
            pl.BlockSpec((1, 128), lambda j: (0, 0)),
            pl.BlockSpec((128, 256), lambda j: (0, 0)),
            pl.BlockSpec((1, 256), lambda j: (0, 0)),
            pl.BlockSpec((1, 256), lambda j: (0, 0)),
            pl.BlockSpec((pr, 32), lambda j: (j, 0)),
            pl.BlockSpec((pr, 64), lambda j: (j, 0)),
            pl.BlockSpec((pr, 128), lambda j: (j, 0)),
            pl.BlockSpec((32, 512), lambda j: (0, 0)),
            pl.BlockSpec((64, 512), lambda j: (0, 0)),
            pl.BlockSpec((128, 512), lambda j: (0, 0)),
            pl.BlockSpec((256, 512), lambda j: (0, 0)),
        ],
        out_specs=[
            pl.BlockSpec((pr, 512), lambda j: (j, 0)),
            pl.BlockSpec((1, 512), lambda j: (0, 0)),
            pl.BlockSpec((1, 512), lambda j: (0, 0)),
        ],
        out_shape=[
            jax.ShapeDtypeStruct((p, 512), jnp.float32),
            jax.ShapeDtypeStruct((1, 512), jnp.float32),
            jax.ShapeDtypeStruct((1, 512), jnp.float32),
        ],
        interpret=_INTERPRET,
    )(h3, sc3, sh3, w4t, sc4, sh4, x1, x2, x3, w5a, w5b, w5c, w5d)


def _out_body(h5_ref, sc_ref, sh_ref, o_ref):
    a = jnp.maximum(h5_ref[...] * sc_ref[...] + sh_ref[...], 0.0)
    o_ref[0] = a.T


def _out_pass(h5, scale, shift, b_, n, pr):
    nb = n // pr
    return pl.pallas_call(
        _out_body,
        grid=(b_, nb),
        in_specs=[
            pl.BlockSpec((pr, 512), lambda b, j, nb=nb: (b * nb + j, 0)),
            pl.BlockSpec((1, 512), lambda b, j: (0, 0)),
            pl.BlockSpec((1, 512), lambda b, j: (0, 0)),
        ],
        out_specs=pl.BlockSpec((1, 512, pr), lambda b, j: (b, 0, j)),
        out_shape=jax.ShapeDtypeStruct((b_, 512, n), jnp.float32),
        interpret=_INTERPRET,
    )(h5, scale, shift)


def _bn_coeffs(s, ss, m, g, b, eps=1e-5):
    mean = s / m
    var = jnp.maximum(ss / m - mean * mean, 0.0)
    scale = g[None, :] / jnp.sqrt(var + eps)
    shift = b[None, :] - mean * scale
    return scale, shift


def kernel(x, W1, W2, W3, W4, W5, g1, b1, g2, b2, g3, b3, g4, b4, g5, b5):
    b_, d_, n = x.shape
    xt = jnp.swapaxes(x, 1, 2)
    t4 = jnp.concatenate(
        [xt, jnp.ones((b_, n, 1), jnp.float32),
         jnp.zeros((b_, n, 1), jnp.float32)], axis=2)
    w1n = W1[:, :2].T
    w1c = W1[:, 2:].T

    rb = 512
    pr = 1024
    p = b_ * n
    m_edge = float(p * KNN)
    m_pt = float(p)

    idx, ptab, qtab = _knn_idx(x, t4, w1n, w1c, rb)
    idx_flat = idx.T.reshape(-1)
    gath = _sc_gather(ptab, idx_flat).reshape(KNN, p, 128)
    h1, s1, ss1 = _conv1(gath, qtab, pr)
    sc1, sh1 = _bn_coeffs(s1, ss1, m_edge, g1, b1)

    x1, h2, s2, ss2 = _stage(h1, sc1, sh1, W2.T, pr)
    sc2, sh2 = _bn_coeffs(s2, ss2, m_edge, g2, b2)

    x2, h3, s3, ss3 = _stage(h2, sc2, sh2, W3.T, pr)
    sc3, sh3 = _bn_coeffs(s3, ss3, m_edge, g3, b3)

    w4t = W4.T
    x3, s4, ss4 = _stage4(h3, sc3, sh3, w4t, pr)
    sc4, sh4 = _bn_coeffs(s4, ss4, m_edge, g4, b4)

    w5t = W5.T
    h5, s5, ss5 = _final_conv(h3, sc3, sh3, w4t, sc4, sh4, x1, x2, x3,
                              w5t[:32], w5t[32:96], w5t[96:224], w5t[224:],
                              pr)
    sc5, sh5 = _bn_coeffs(s5, ss5, m_pt, g5, b5)

    return _out_pass(h5, sc5, sh5, b_, n, pr)
